# Initial kernel scaffold; baseline (speedup 1.0000x reference)
import functools

import jax
import jax.numpy as jnp
from jax import lax
from jax.experimental import pallas as pl
from jax.experimental.pallas import tpu as pltpu
from jax.experimental.pallas import tpu_sc as plsc

N = 10000
E = 320000
D = 128
H = 128

NC = 2
NS = 16
NW = NC * NS
EPW = E // NW
CH = 80
NCHUNK = EPW // CH
ROWS_PER_TILE = N // NS


def _pre_body(x_ref, wsrc_ref, wdst_ref, b_ref, psrc_ref, pdst_ref):
    x = x_ref[...]
    dn = (((1,), (1,)), ((), ()))
    psrc_ref[...] = lax.dot_general(x, wsrc_ref[...], dn,
                                    preferred_element_type=jnp.float32)
    pdst_ref[...] = lax.dot_general(x, wdst_ref[...], dn,
                                    preferred_element_type=jnp.float32) + b_ref[...]


def _precompute(x, wsrc, wdst, b_msg):
    grid = (10,)
    bn = N // 10
    return pl.pallas_call(
        _pre_body,
        grid=grid,
        in_specs=[
            pl.BlockSpec((bn, D), lambda i: (i, 0)),
            pl.BlockSpec((H, D), lambda i: (0, 0)),
            pl.BlockSpec((H, D), lambda i: (0, 0)),
            pl.BlockSpec((1, H), lambda i: (0, 0)),
        ],
        out_specs=[
            pl.BlockSpec((bn, H), lambda i: (i, 0)),
            pl.BlockSpec((bn, H), lambda i: (i, 0)),
        ],
        out_shape=[
            jax.ShapeDtypeStruct((N, H), jnp.float32),
            jax.ShapeDtypeStruct((N, H), jnp.float32),
        ],
    )(x, wsrc, wdst, b_msg.reshape(1, H))


def _sc_edge_kernel(psrc_hbm, pdst_hbm, pos_hbm, row_hbm, col_hbm, wd_hbm,
                    zeros_hbm, out_hbm,
                    row_v, col_v, src_v, dst_v, posi_v, posj_v, wd_v,
                    aggr_sh, sem1, sem2, sem3, sem4):
    c = lax.axis_index("c")
    s = lax.axis_index("s")
    wid = s * NC + c

    pltpu.sync_copy(zeros_hbm.at[pl.ds(s * ROWS_PER_TILE, ROWS_PER_TILE)],
                    aggr_sh.at[pl.ds(s * ROWS_PER_TILE, ROWS_PER_TILE)])
    pltpu.sync_copy(wd_hbm, wd_v)
    plsc.subcore_barrier()

    wk = tuple(wd_v[pl.ds(k * 16, 16)] for k in range(H // 16))

    def chunk_body(ch, carry):
        base = wid * EPW + ch * CH
        pltpu.sync_copy(row_hbm.at[pl.ds(base, CH)], row_v)
        pltpu.sync_copy(col_hbm.at[pl.ds(base, CH)], col_v)
        cp1 = pltpu.async_copy(psrc_hbm.at[row_v], src_v, sem1)
        cp2 = pltpu.async_copy(pdst_hbm.at[col_v], dst_v, sem2)
        cp3 = pltpu.async_copy(pos_hbm.at[row_v], posi_v, sem3)
        cp4 = pltpu.async_copy(pos_hbm.at[col_v], posj_v, sem4)
        cp1.wait()
        cp2.wait()
        cp3.wait()
        cp4.wait()

        def edge_body(e, ecarry):
            dv = posi_v[e] - posj_v[e]
            dist = jnp.sum(dv * dv)
            for k in range(H // 16):
                sl = pl.ds(k * 16, 16)
                m = src_v[e, sl] + dst_v[e, sl] + dist * wk[k]
                src_v[e, sl] = jnp.maximum(m, 0.0)
            return ecarry

        lax.fori_loop(0, CH, edge_body, 0)
        pltpu.sync_copy(src_v, aggr_sh.at[col_v], add=True)
        return carry

    lax.fori_loop(0, NCHUNK, chunk_body, 0)
    plsc.subcore_barrier()
    pltpu.sync_copy(aggr_sh.at[pl.ds(s * ROWS_PER_TILE, ROWS_PER_TILE)],
                    out_hbm.at[c, pl.ds(s * ROWS_PER_TILE, ROWS_PER_TILE)])


def _sc_aggregate(psrc, pdst, pos_pad, row, col, w_d, zeros):
    mesh = plsc.VectorSubcoreMesh(core_axis_name="c", subcore_axis_name="s")
    fn = pl.kernel(
        _sc_edge_kernel, mesh=mesh,
        out_type=jax.ShapeDtypeStruct((NC, N, H), jnp.float32),
        scratch_types=[
            pltpu.VMEM((CH,), jnp.int32),
            pltpu.VMEM((CH,), jnp.int32),
            pltpu.VMEM((CH, H), jnp.float32),
            pltpu.VMEM((CH, H), jnp.float32),
            pltpu.VMEM((CH, 16), jnp.float32),
            pltpu.VMEM((CH, 16), jnp.float32),
            pltpu.VMEM((H,), jnp.float32),
            pltpu.VMEM_SHARED((N, H), jnp.float32),
            pltpu.SemaphoreType.DMA,
            pltpu.SemaphoreType.DMA,
            pltpu.SemaphoreType.DMA,
            pltpu.SemaphoreType.DMA,
        ],
    )
    return fn(psrc, pdst, pos_pad, row, col, w_d, zeros)


def _post_body(x_ref, a0_ref, a1_ref, wres_ref, w1_ref, w2_ref, b_ref, o_ref):
    x = x_ref[...]
    a = a0_ref[...] + a1_ref[...]
    dn = (((1,), (1,)), ((), ()))
    h = (lax.dot_general(x, w1_ref[...], dn, preferred_element_type=jnp.float32)
         + lax.dot_general(a, w2_ref[...], dn, preferred_element_type=jnp.float32)
         + b_ref[...])
    o_ref[...] = (lax.dot_general(x, wres_ref[...], dn,
                                  preferred_element_type=jnp.float32)
                  + jnp.maximum(h, 0.0))


def _update(x, a0, a1, wres, w1, w2, b_upd):
    grid = (10,)
    bn = N // 10
    return pl.pallas_call(
        _post_body,
        grid=grid,
        in_specs=[
            pl.BlockSpec((bn, D), lambda i: (i, 0)),
            pl.BlockSpec((bn, H), lambda i: (i, 0)),
            pl.BlockSpec((bn, H), lambda i: (i, 0)),
            pl.BlockSpec((H, D), lambda i: (0, 0)),
            pl.BlockSpec((H, D), lambda i: (0, 0)),
            pl.BlockSpec((H, H), lambda i: (0, 0)),
            pl.BlockSpec((1, H), lambda i: (0, 0)),
        ],
        out_specs=pl.BlockSpec((bn, H), lambda i: (i, 0)),
        out_shape=jax.ShapeDtypeStruct((N, H), jnp.float32),
    )(x, a0, a1, wres, w1, w2, b_upd.reshape(1, H))


def kernel(node_embed, node_pos, edge_index, W_res, W_msg, b_msg, W_upd, b_upd):
    row = edge_index[0]
    col = edge_index[1]
    wsrc = W_msg[:, :D]
    wdst = W_msg[:, D:2 * D]
    w_d = W_msg[:, 2 * D]
    w1 = W_upd[:, :D]
    w2 = W_upd[:, D:]
    pos_pad = jnp.pad(node_pos, ((0, 0), (0, 13)))
    zeros = jnp.zeros((N, H), jnp.float32)

    psrc, pdst = _precompute(node_embed, wsrc, wdst, b_msg)
    partials = _sc_aggregate(psrc, pdst, pos_pad, row, col, w_d, zeros)
    return _update(node_embed, partials[0], partials[1], W_res, w1, w2, b_upd)


# speedup vs baseline: 3.7576x; 3.7576x over previous
import functools

import jax
import jax.numpy as jnp
from jax import lax
from jax.experimental import pallas as pl
from jax.experimental.pallas import tpu as pltpu
from jax.experimental.pallas import tpu_sc as plsc

N = 10000
E = 320000
D = 128
H = 128

NC = 2
NS = 16
NW = NC * NS
EPW = E // NW
CH = 80
NCHUNK = EPW // CH
NPAD = 10240
ROWS_PER_TILE = NPAD // NS


def _pre_body(x_ref, wsrc_ref, wdst_ref, b_ref, psrc_ref, pdst_ref):
    x = x_ref[...]
    dn = (((1,), (1,)), ((), ()))
    psrc_ref[...] = lax.dot_general(x, wsrc_ref[...], dn,
                                    preferred_element_type=jnp.float32)
    pdst_ref[...] = lax.dot_general(x, wdst_ref[...], dn,
                                    preferred_element_type=jnp.float32) + b_ref[...]


def _precompute(x, wsrc, wdst, b_msg):
    grid = (10,)
    bn = N // 10
    return pl.pallas_call(
        _pre_body,
        grid=grid,
        in_specs=[
            pl.BlockSpec((bn, D), lambda i: (i, 0)),
            pl.BlockSpec((H, D), lambda i: (0, 0)),
            pl.BlockSpec((H, D), lambda i: (0, 0)),
            pl.BlockSpec((1, H), lambda i: (0, 0)),
        ],
        out_specs=[
            pl.BlockSpec((bn, H), lambda i: (i, 0)),
            pl.BlockSpec((bn, H), lambda i: (i, 0)),
        ],
        out_shape=[
            jax.ShapeDtypeStruct((N, H), jnp.float32),
            jax.ShapeDtypeStruct((N, H), jnp.float32),
        ],
    )(x, wsrc, wdst, b_msg.reshape(1, H))


def _sc_edge_kernel(psrc_hbm, pdst_hbm, pos_hbm, row_hbm, col_hbm, wd_hbm,
                    zeros_hbm, out_hbm,
                    row_v, col_v, src_v, dst_v, posi_v, posj_v, wd_v,
                    aggr_sh, sem1, sem2, sem3, sem4):
    c = lax.axis_index("c")
    s = lax.axis_index("s")
    wid = s * NC + c
    stripe = pl.multiple_of(s * ROWS_PER_TILE, 8)

    pltpu.sync_copy(zeros_hbm.at[pl.ds(stripe, ROWS_PER_TILE)],
                    aggr_sh.at[pl.ds(stripe, ROWS_PER_TILE)])
    pltpu.sync_copy(wd_hbm, wd_v)
    plsc.subcore_barrier()

    wk = tuple(wd_v[pl.ds(k * 16, 16)] for k in range(H // 16))

    def chunk_body(ch, carry):
        base = pl.multiple_of(wid * EPW + ch * CH, 8)
        pltpu.sync_copy(row_hbm.at[pl.ds(base, CH)], row_v)
        pltpu.sync_copy(col_hbm.at[pl.ds(base, CH)], col_v)
        cp1 = pltpu.async_copy(psrc_hbm.at[row_v], src_v, sem1)
        cp2 = pltpu.async_copy(pdst_hbm.at[col_v], dst_v, sem2)
        cp3 = pltpu.async_copy(pos_hbm.at[row_v], posi_v, sem3)
        cp4 = pltpu.async_copy(pos_hbm.at[col_v], posj_v, sem4)
        cp1.wait()
        cp2.wait()
        cp3.wait()
        cp4.wait()

        def edge_body(e, ecarry):
            dv = posi_v[e] - posj_v[e]
            dv2 = dv * dv
            dist = dv2[0] + dv2[1] + dv2[2]
            for k in range(H // 16):
                sl = pl.ds(k * 16, 16)
                m = src_v[e, sl] + dst_v[e, sl] + dist * wk[k]
                src_v[e, sl] = jnp.maximum(m, 0.0)
            return ecarry

        lax.fori_loop(0, CH, edge_body, 0)
        pltpu.sync_copy(src_v, aggr_sh.at[col_v], add=True)
        return carry

    lax.fori_loop(0, NCHUNK, chunk_body, 0)
    plsc.subcore_barrier()
    pltpu.sync_copy(aggr_sh.at[pl.ds(stripe, ROWS_PER_TILE)],
                    out_hbm.at[c, pl.ds(stripe, ROWS_PER_TILE)])


def _sc_aggregate(psrc, pdst, pos_pad, row, col, w_d, zeros):
    mesh = plsc.VectorSubcoreMesh(core_axis_name="c", subcore_axis_name="s")
    fn = pl.kernel(
        _sc_edge_kernel, mesh=mesh,
        out_type=jax.ShapeDtypeStruct((NC, NPAD, H), jnp.float32),
        scratch_types=[
            pltpu.VMEM((CH,), jnp.int32),
            pltpu.VMEM((CH,), jnp.int32),
            pltpu.VMEM((CH, H), jnp.float32),
            pltpu.VMEM((CH, H), jnp.float32),
            pltpu.VMEM((CH, 16), jnp.float32),
            pltpu.VMEM((CH, 16), jnp.float32),
            pltpu.VMEM((H,), jnp.float32),
            pltpu.VMEM_SHARED((NPAD, H), jnp.float32),
            pltpu.SemaphoreType.DMA,
            pltpu.SemaphoreType.DMA,
            pltpu.SemaphoreType.DMA,
            pltpu.SemaphoreType.DMA,
        ],
        compiler_params=pltpu.CompilerParams(use_tc_tiling_on_sc=False),
    )
    return fn(psrc, pdst, pos_pad, row, col, w_d, zeros)


def _post_body(x_ref, a0_ref, a1_ref, wres_ref, w1_ref, w2_ref, b_ref, o_ref):
    x = x_ref[...]
    a = a0_ref[...] + a1_ref[...]
    dn = (((1,), (1,)), ((), ()))
    h = (lax.dot_general(x, w1_ref[...], dn, preferred_element_type=jnp.float32)
         + lax.dot_general(a, w2_ref[...], dn, preferred_element_type=jnp.float32)
         + b_ref[...])
    o_ref[...] = (lax.dot_general(x, wres_ref[...], dn,
                                  preferred_element_type=jnp.float32)
                  + jnp.maximum(h, 0.0))


def _update(x, a0, a1, wres, w1, w2, b_upd):
    grid = (10,)
    bn = N // 10
    return pl.pallas_call(
        _post_body,
        grid=grid,
        in_specs=[
            pl.BlockSpec((bn, D), lambda i: (i, 0)),
            pl.BlockSpec((bn, H), lambda i: (i, 0)),
            pl.BlockSpec((bn, H), lambda i: (i, 0)),
            pl.BlockSpec((H, D), lambda i: (0, 0)),
            pl.BlockSpec((H, D), lambda i: (0, 0)),
            pl.BlockSpec((H, H), lambda i: (0, 0)),
            pl.BlockSpec((1, H), lambda i: (0, 0)),
        ],
        out_specs=pl.BlockSpec((bn, H), lambda i: (i, 0)),
        out_shape=jax.ShapeDtypeStruct((N, H), jnp.float32),
    )(x, a0, a1, wres, w1, w2, b_upd.reshape(1, H))


def kernel(node_embed, node_pos, edge_index, W_res, W_msg, b_msg, W_upd, b_upd):
    row = edge_index[0]
    col = edge_index[1]
    wsrc = W_msg[:, :D]
    wdst = W_msg[:, D:2 * D]
    w_d = W_msg[:, 2 * D]
    w1 = W_upd[:, :D]
    w2 = W_upd[:, D:]
    pos_pad = jnp.pad(node_pos, ((0, 0), (0, 13)))
    zeros = jnp.zeros((NPAD, H), jnp.float32)

    psrc, pdst = _precompute(node_embed, wsrc, wdst, b_msg)
    partials = _sc_aggregate(psrc, pdst, pos_pad, row, col, w_d, zeros)
    return _update(node_embed, partials[0], partials[1], W_res, w1, w2, b_upd)
